# baseline (speedup 1.0000x reference)
import functools

import jax
import jax.numpy as jnp
import numpy as np
from jax import lax
from jax.experimental import pallas as pl
from jax.experimental.pallas import tpu as pltpu
from jax.experimental.pallas import tpu_sc as plsc

B = 4096
D = 128
L = 20
GN = 1000
NC = 2
NS = 16
NW = NC * NS
BPW = B // NW
NBUF = 4


def _sc_embed(user_i, item_i, country_i, gflat, tflat, scat, zeros,
              user_table, item_table, genre_table, country_table, tags_table):
    mesh = plsc.VectorSubcoreMesh(core_axis_name="c", subcore_axis_name="s")
    emb_ty = jax.ShapeDtypeStruct((B, D), jnp.float32)

    @functools.partial(
        pl.kernel,
        mesh=mesh,
        out_type=(emb_ty, emb_ty, emb_ty, emb_ty, emb_ty),
        scratch_types=[
            pltpu.VMEM((BPW,), jnp.int32),
            pltpu.VMEM((L * BPW,), jnp.int32),
            pltpu.VMEM((L, BPW), jnp.int32),
            pltpu.VMEM((BPW, D), jnp.float32),
            pltpu.VMEM((NBUF, BPW, D), jnp.float32),
            pltpu.VMEM_SHARED((2, NS, BPW, D), jnp.float32),
            pltpu.VMEM_SHARED((GN, D), jnp.float32),
            pltpu.SemaphoreType.DMA,
            pltpu.SemaphoreType.DMA((NBUF,)),
            pltpu.SemaphoreType.DMA((NBUF,)),
            pltpu.SemaphoreType.DMA((2,)),
            pltpu.SemaphoreType.DMA,
        ],
    )
    def sc_kernel(u_hbm, i_hbm, c_hbm, gf_hbm, tf_hbm, sc_hbm, z_hbm,
                  ut_hbm, it_hbm, gt_hbm, ct_hbm, tt_hbm,
                  uo_hbm, io_hbm, go_hbm, co_hbm, to_hbm,
                  idx1_v, idxm_v, scat_v, rows_v, ring_v, acc_sh, gtab_sh,
                  sem, gsem, ssem, zsem, tsem):
        c = lax.axis_index("c")
        s = lax.axis_index("s")
        wid = c * NS + s
        base = wid * BPW

        slab_g = acc_sh.at[0].at[s]
        slab_t = acc_sh.at[1].at[s]
        pltpu.make_async_copy(z_hbm, slab_g, zsem.at[0]).start()
        pltpu.make_async_copy(z_hbm, slab_t, zsem.at[1]).start()

        @pl.when(s == 0)
        def _():
            pltpu.make_async_copy(gt_hbm, gtab_sh, tsem).start()

        def single(idx_hbm, tab_hbm, out_hbm):
            pltpu.sync_copy(idx_hbm.at[pl.ds(base, BPW)], idx1_v)
            pltpu.sync_copy(tab_hbm.at[idx1_v], rows_v)
            pltpu.sync_copy(rows_v, out_hbm.at[pl.ds(base, BPW)])

        single(u_hbm, ut_hbm, uo_hbm)
        single(i_hbm, it_hbm, io_hbm)
        single(c_hbm, ct_hbm, co_hbm)

        pltpu.sync_copy(sc_hbm, scat_v)

        def multi(flat_hbm, tab_hbm, out_hbm, slab, ztbl):
            def gather(g, buf):
                return pltpu.make_async_copy(
                    tab_hbm.at[idxm_v.at[pl.ds(g * BPW, BPW)]],
                    ring_v.at[buf], gsem.at[buf])

            def scatter(g, buf):
                return pltpu.make_async_copy(
                    ring_v.at[buf], slab.at[scat_v.at[g]], ssem.at[buf])

            pltpu.sync_copy(flat_hbm.at[pl.ds(base * L, L * BPW)], idxm_v)
            pltpu.make_async_copy(z_hbm, slab, zsem.at[ztbl]).wait()

            for g in range(NBUF - 1):
                gather(g, g).start()

            @pl.loop(0, L)
            def _(g):
                buf = lax.rem(g, NBUF)
                gather(g, buf).wait()
                scatter(g, buf).start(add=True)
                nxt = g + NBUF - 1

                @pl.when(nxt < L)
                def _():
                    nbuf = lax.rem(nxt, NBUF)
                    @pl.when(nxt >= NBUF)
                    def _():
                        scatter(nxt - NBUF, nbuf).wait()
                    gather(nxt, nbuf).start()

            for g in range(L - NBUF, L):
                scatter(g, g % NBUF).wait()
            pltpu.sync_copy(slab, out_hbm.at[pl.ds(base, BPW)])

        multi(tf_hbm, tt_hbm, to_hbm, slab_t, 1)

        @pl.when(s == 0)
        def _():
            pltpu.make_async_copy(gt_hbm, gtab_sh, tsem).wait()

        plsc.subcore_barrier()
        multi(gf_hbm, gtab_sh, go_hbm, slab_g, 0)

    return sc_kernel(user_i, item_i, country_i, gflat, tflat, scat, zeros,
                     user_table, item_table, genre_table, country_table,
                     tags_table)


TB = 512


def _mlp_body(u_ref, i_ref, g_ref, c_ref, t_ref,
              w1_ref, b1_ref, w2_ref, b2_ref, w3_ref, b3_ref, o_ref):
    embs = (u_ref[...], i_ref[...], g_ref[...], c_ref[...], t_ref[...])
    h = b1_ref[...]
    for idx, e in enumerate(embs):
        h = h + jnp.dot(e, w1_ref[idx], preferred_element_type=jnp.float32)
    h = jnp.maximum(h, 0.0)
    h2 = jnp.dot(h, w2_ref[...], preferred_element_type=jnp.float32)
    h2 = jnp.maximum(h2 + b2_ref[...], 0.0)
    o = jnp.dot(h2, w3_ref[...], preferred_element_type=jnp.float32)
    o_ref[...] = o + b3_ref[...]


def _mlp(u, i, g, c, t, W1, b1, W2, b2, W3, b3):
    w1r = W1.reshape(5, D, D)
    full = lambda shape: pl.BlockSpec(shape, lambda n: tuple(0 for _ in shape))
    out = pl.pallas_call(
        _mlp_body,
        grid=(B // TB,),
        in_specs=[
            pl.BlockSpec((TB, D), lambda n: (n, 0)),
            pl.BlockSpec((TB, D), lambda n: (n, 0)),
            pl.BlockSpec((TB, D), lambda n: (n, 0)),
            pl.BlockSpec((TB, D), lambda n: (n, 0)),
            pl.BlockSpec((TB, D), lambda n: (n, 0)),
            full((5, D, D)),
            full((1, D)),
            full((D, D // 2)),
            full((1, D // 2)),
            full((D // 2, 1)),
            full((1, 1)),
        ],
        out_specs=pl.BlockSpec((TB, 1), lambda n: (n, 0)),
        out_shape=jax.ShapeDtypeStruct((B, 1), jnp.float32),
    )(u, i, g, c, t, w1r, b1.reshape(1, D), W2, b2.reshape(1, D // 2),
      W3, b3.reshape(1, 1))
    return out.reshape(-1)


_SCAT = (np.arange(L * BPW) // L).astype(np.int32).reshape(L, BPW)


def kernel(user, item, genre, country, tags,
           user_table, item_table, genre_table, country_table, tags_table,
           W1, b1, W2, b2, W3, b3):
    user_i = user.astype(jnp.int32)
    item_i = item.astype(jnp.int32)
    country_i = country.astype(jnp.int32)
    gflat = genre.astype(jnp.int32).reshape(-1)
    tflat = tags.astype(jnp.int32).reshape(-1)
    scat = jnp.asarray(_SCAT)
    zeros = jnp.zeros((BPW, D), jnp.float32)

    u_e, i_e, g_e, c_e, t_e = _sc_embed(
        user_i, item_i, country_i, gflat, tflat, scat, zeros,
        user_table, item_table, genre_table, country_table, tags_table)

    return _mlp(u_e, i_e, g_e, c_e, t_e, W1, b1, W2, b2, W3, b3)


# speedup vs baseline: 1.1236x; 1.1236x over previous
import functools

import jax
import jax.numpy as jnp
import numpy as np
from jax import lax
from jax.experimental import pallas as pl
from jax.experimental.pallas import tpu as pltpu
from jax.experimental.pallas import tpu_sc as plsc

B = 4096
D = 128
L = 20
NC = 2
NS = 16
NW = NC * NS
BPW = B // NW
NBUF = 4


def _sc_embed(user_i, item_i, country_i, gflat, tflat, scat, zeros,
              user_table, item_table, genre_table, country_table, tags_table):
    mesh = plsc.VectorSubcoreMesh(core_axis_name="c", subcore_axis_name="s")
    emb_ty = jax.ShapeDtypeStruct((B, D), jnp.float32)

    @functools.partial(
        pl.kernel,
        mesh=mesh,
        out_type=(emb_ty, emb_ty, emb_ty, emb_ty, emb_ty),
        scratch_types=[
            pltpu.VMEM((BPW,), jnp.int32),
            pltpu.VMEM((L * BPW,), jnp.int32),
            pltpu.VMEM((8, BPW), jnp.int32),
            pltpu.VMEM((BPW, D), jnp.float32),
            pltpu.VMEM((NBUF, BPW, D), jnp.float32),
            pltpu.VMEM_SHARED((2, NS, BPW, D), jnp.float32),
            pltpu.SemaphoreType.DMA,
            pltpu.SemaphoreType.DMA((NBUF,)),
            pltpu.SemaphoreType.DMA((NBUF,)),
            pltpu.SemaphoreType.DMA((2,)),
        ],
    )
    def sc_kernel(u_hbm, i_hbm, c_hbm, gf_hbm, tf_hbm, sc_hbm, z_hbm,
                  ut_hbm, it_hbm, gt_hbm, ct_hbm, tt_hbm,
                  uo_hbm, io_hbm, go_hbm, co_hbm, to_hbm,
                  idx1_v, idxm_v, scat_v, rows_v, ring_v, acc_sh,
                  sem, gsem, ssem, zsem):
        c = lax.axis_index("c")
        s = lax.axis_index("s")
        wid = c * NS + s
        base = wid * BPW

        slab_g = acc_sh.at[0].at[s]
        slab_t = acc_sh.at[1].at[s]
        pltpu.make_async_copy(z_hbm, slab_g, zsem.at[0]).start()
        pltpu.make_async_copy(z_hbm, slab_t, zsem.at[1]).start()

        def single(idx_hbm, tab_hbm, out_hbm):
            pltpu.sync_copy(idx_hbm.at[pl.ds(base, BPW)], idx1_v)
            pltpu.sync_copy(tab_hbm.at[idx1_v], rows_v)
            pltpu.sync_copy(rows_v, out_hbm.at[pl.ds(base, BPW)])

        single(u_hbm, ut_hbm, uo_hbm)
        single(i_hbm, it_hbm, io_hbm)
        single(c_hbm, ct_hbm, co_hbm)

        pltpu.sync_copy(sc_hbm, scat_v)

        def multi(flat_hbm, tab_hbm, out_hbm, slab, ztbl):
            def gather(g, buf):
                return pltpu.make_async_copy(
                    tab_hbm.at[idxm_v.at[pl.ds(g * BPW, BPW)]],
                    ring_v.at[buf], gsem.at[buf])

            def scatter(g, buf):
                return pltpu.make_async_copy(
                    ring_v.at[buf], slab.at[scat_v.at[0]], ssem.at[buf])

            pltpu.sync_copy(flat_hbm.at[pl.ds(base * L, L * BPW)], idxm_v)
            pltpu.make_async_copy(z_hbm, slab, zsem.at[ztbl]).wait()

            for g in range(NBUF - 1):
                gather(g, g).start()

            @pl.loop(0, L)
            def _(g):
                buf = lax.rem(g, NBUF)
                gather(g, buf).wait()
                scatter(g, buf).start(add=True)
                nxt = g + NBUF - 1

                @pl.when(nxt < L)
                def _():
                    nbuf = lax.rem(nxt, NBUF)
                    @pl.when(nxt >= NBUF)
                    def _():
                        scatter(nxt - NBUF, nbuf).wait()
                    gather(nxt, nbuf).start()

            for g in range(L - NBUF, L):
                scatter(g, g % NBUF).wait()
            pltpu.sync_copy(slab, out_hbm.at[pl.ds(base, BPW)])

        multi(gf_hbm, gt_hbm, go_hbm, slab_g, 0)
        multi(tf_hbm, tt_hbm, to_hbm, slab_t, 1)

    return sc_kernel(user_i, item_i, country_i, gflat, tflat, scat, zeros,
                     user_table, item_table, genre_table, country_table,
                     tags_table)


TB = 512


def _mlp_body(u_ref, i_ref, g_ref, c_ref, t_ref,
              w1_ref, b1_ref, w2_ref, b2_ref, w3_ref, b3_ref, o_ref):
    embs = (u_ref[...], i_ref[...], g_ref[...], c_ref[...], t_ref[...])
    h = b1_ref[...]
    for idx, e in enumerate(embs):
        h = h + jnp.dot(e, w1_ref[idx], preferred_element_type=jnp.float32)
    h = jnp.maximum(h, 0.0)
    h2 = jnp.dot(h, w2_ref[...], preferred_element_type=jnp.float32)
    h2 = jnp.maximum(h2 + b2_ref[...], 0.0)
    o = jnp.dot(h2, w3_ref[...], preferred_element_type=jnp.float32)
    o_ref[...] = o + b3_ref[...]


def _mlp(u, i, g, c, t, W1, b1, W2, b2, W3, b3):
    w1r = W1.reshape(5, D, D)
    full = lambda shape: pl.BlockSpec(shape, lambda n: tuple(0 for _ in shape))
    out = pl.pallas_call(
        _mlp_body,
        grid=(B // TB,),
        in_specs=[
            pl.BlockSpec((TB, D), lambda n: (n, 0)),
            pl.BlockSpec((TB, D), lambda n: (n, 0)),
            pl.BlockSpec((TB, D), lambda n: (n, 0)),
            pl.BlockSpec((TB, D), lambda n: (n, 0)),
            pl.BlockSpec((TB, D), lambda n: (n, 0)),
            full((5, D, D)),
            full((1, D)),
            full((D, D // 2)),
            full((1, D // 2)),
            full((D // 2, 1)),
            full((1, 1)),
        ],
        out_specs=pl.BlockSpec((TB, 1), lambda n: (n, 0)),
        out_shape=jax.ShapeDtypeStruct((B, 1), jnp.float32),
    )(u, i, g, c, t, w1r, b1.reshape(1, D), W2, b2.reshape(1, D // 2),
      W3, b3.reshape(1, 1))
    return out.reshape(-1)


_SCAT = np.tile(np.arange(BPW, dtype=np.int32), (8, 1))


def kernel(user, item, genre, country, tags,
           user_table, item_table, genre_table, country_table, tags_table,
           W1, b1, W2, b2, W3, b3):
    user_i = user.astype(jnp.int32)
    item_i = item.astype(jnp.int32)
    country_i = country.astype(jnp.int32)
    gflat = genre.astype(jnp.int32).reshape(NW, BPW, L).transpose(0, 2, 1).reshape(-1)
    tflat = tags.astype(jnp.int32).reshape(NW, BPW, L).transpose(0, 2, 1).reshape(-1)
    scat = jnp.asarray(_SCAT)
    zeros = jnp.zeros((BPW, D), jnp.float32)

    u_e, i_e, g_e, c_e, t_e = _sc_embed(
        user_i, item_i, country_i, gflat, tflat, scat, zeros,
        user_table, item_table, genre_table, country_table, tags_table)

    return _mlp(u_e, i_e, g_e, c_e, t_e, W1, b1, W2, b2, W3, b3)


# speedup vs baseline: 1.5332x; 1.3645x over previous
import functools

import jax
import jax.numpy as jnp
import numpy as np
from jax import lax
from jax.experimental import pallas as pl
from jax.experimental.pallas import tpu as pltpu
from jax.experimental.pallas import tpu_sc as plsc

B = 4096
D = 128
L = 20
GN = 1000
NC = 2
NS = 16
NW = NC * NS
BPW = B // NW
NBUF = 4


def _sc_embed(user_i, item_i, country_i, tflat, scat, zeros,
              user_table, item_table, country_table, tags_table):
    mesh = plsc.VectorSubcoreMesh(core_axis_name="c", subcore_axis_name="s")
    emb_ty = jax.ShapeDtypeStruct((B, D), jnp.float32)

    @functools.partial(
        pl.kernel,
        mesh=mesh,
        out_type=(emb_ty, emb_ty, emb_ty, emb_ty),
        scratch_types=[
            pltpu.VMEM((BPW,), jnp.int32),
            pltpu.VMEM((L * BPW,), jnp.int32),
            pltpu.VMEM((8, BPW), jnp.int32),
            pltpu.VMEM((BPW, D), jnp.float32),
            pltpu.VMEM((NBUF, BPW, D), jnp.float32),
            pltpu.VMEM_SHARED((NS, BPW, D), jnp.float32),
            pltpu.SemaphoreType.DMA((NBUF,)),
            pltpu.SemaphoreType.DMA((NBUF,)),
            pltpu.SemaphoreType.DMA,
        ],
    )
    def sc_kernel(u_hbm, i_hbm, c_hbm, tf_hbm, sc_hbm, z_hbm,
                  ut_hbm, it_hbm, ct_hbm, tt_hbm,
                  uo_hbm, io_hbm, co_hbm, to_hbm,
                  idx1_v, idxm_v, scat_v, rows_v, ring_v, acc_sh,
                  gsem, ssem, zsem):
        c = lax.axis_index("c")
        s = lax.axis_index("s")
        wid = c * NS + s
        base = wid * BPW

        slab = acc_sh.at[s]
        pltpu.make_async_copy(z_hbm, slab, zsem).start()

        def single(idx_hbm, tab_hbm, out_hbm):
            pltpu.sync_copy(idx_hbm.at[pl.ds(base, BPW)], idx1_v)
            pltpu.sync_copy(tab_hbm.at[idx1_v], rows_v)
            pltpu.sync_copy(rows_v, out_hbm.at[pl.ds(base, BPW)])

        single(u_hbm, ut_hbm, uo_hbm)
        single(i_hbm, it_hbm, io_hbm)
        single(c_hbm, ct_hbm, co_hbm)

        pltpu.sync_copy(sc_hbm, scat_v)

        def gather(g, buf):
            return pltpu.make_async_copy(
                tt_hbm.at[idxm_v.at[pl.ds(g * BPW, BPW)]],
                ring_v.at[buf], gsem.at[buf])

        def scatter(g, buf):
            return pltpu.make_async_copy(
                ring_v.at[buf], slab.at[scat_v.at[0]], ssem.at[buf])

        pltpu.sync_copy(tf_hbm.at[pl.ds(base * L, L * BPW)], idxm_v)
        pltpu.make_async_copy(z_hbm, slab, zsem).wait()

        for g in range(NBUF - 1):
            gather(g, g).start()

        @pl.loop(0, L)
        def _(g):
            buf = lax.rem(g, NBUF)
            gather(g, buf).wait()
            scatter(g, buf).start(add=True)
            nxt = g + NBUF - 1

            @pl.when(nxt < L)
            def _():
                nbuf = lax.rem(nxt, NBUF)
                @pl.when(nxt >= NBUF)
                def _():
                    scatter(nxt - NBUF, nbuf).wait()
                gather(nxt, nbuf).start()

        for g in range(L - NBUF, L):
            scatter(g, g % NBUF).wait()
        pltpu.sync_copy(slab, to_hbm.at[pl.ds(base, BPW)])

    return sc_kernel(user_i, item_i, country_i, tflat, scat, zeros,
                     user_table, item_table, country_table, tags_table)


GTB = 512


def _genre_body(g_ref, gt_ref, o_ref):
    gi = g_ref[...]
    vio = lax.broadcasted_iota(jnp.int32, (GTB, GN), 1)
    counts = (gi[:, 0:1] == vio).astype(jnp.float32)
    for l in range(1, L):
        counts = counts + (gi[:, l:l + 1] == vio).astype(jnp.float32)
    o_ref[...] = jnp.dot(counts, gt_ref[...],
                         preferred_element_type=jnp.float32)


def _genre_emb(genre_i, genre_table):
    return pl.pallas_call(
        _genre_body,
        grid=(B // GTB,),
        in_specs=[
            pl.BlockSpec((GTB, L), lambda n: (n, 0)),
            pl.BlockSpec((GN, D), lambda n: (0, 0)),
        ],
        out_specs=pl.BlockSpec((GTB, D), lambda n: (n, 0)),
        out_shape=jax.ShapeDtypeStruct((B, D), jnp.float32),
    )(genre_i, genre_table)


TB = 512


def _mlp_body(u_ref, i_ref, g_ref, c_ref, t_ref,
              w1_ref, b1_ref, w2_ref, b2_ref, w3_ref, b3_ref, o_ref):
    embs = (u_ref[...], i_ref[...], g_ref[...], c_ref[...], t_ref[...])
    h = b1_ref[...]
    for idx, e in enumerate(embs):
        h = h + jnp.dot(e, w1_ref[idx], preferred_element_type=jnp.float32)
    h = jnp.maximum(h, 0.0)
    h2 = jnp.dot(h, w2_ref[...], preferred_element_type=jnp.float32)
    h2 = jnp.maximum(h2 + b2_ref[...], 0.0)
    o = jnp.dot(h2, w3_ref[...], preferred_element_type=jnp.float32)
    o_ref[...] = o + b3_ref[...]


def _mlp(u, i, g, c, t, W1, b1, W2, b2, W3, b3):
    w1r = W1.reshape(5, D, D)
    full = lambda shape: pl.BlockSpec(shape, lambda n: tuple(0 for _ in shape))
    out = pl.pallas_call(
        _mlp_body,
        grid=(B // TB,),
        in_specs=[
            pl.BlockSpec((TB, D), lambda n: (n, 0)),
            pl.BlockSpec((TB, D), lambda n: (n, 0)),
            pl.BlockSpec((TB, D), lambda n: (n, 0)),
            pl.BlockSpec((TB, D), lambda n: (n, 0)),
            pl.BlockSpec((TB, D), lambda n: (n, 0)),
            full((5, D, D)),
            full((1, D)),
            full((D, D // 2)),
            full((1, D // 2)),
            full((D // 2, 1)),
            full((1, 1)),
        ],
        out_specs=pl.BlockSpec((TB, 1), lambda n: (n, 0)),
        out_shape=jax.ShapeDtypeStruct((B, 1), jnp.float32),
    )(u, i, g, c, t, w1r, b1.reshape(1, D), W2, b2.reshape(1, D // 2),
      W3, b3.reshape(1, 1))
    return out.reshape(-1)


_SCAT = np.tile(np.arange(BPW, dtype=np.int32), (8, 1))


def kernel(user, item, genre, country, tags,
           user_table, item_table, genre_table, country_table, tags_table,
           W1, b1, W2, b2, W3, b3):
    user_i = user.astype(jnp.int32)
    item_i = item.astype(jnp.int32)
    country_i = country.astype(jnp.int32)
    genre_i = genre.astype(jnp.int32)
    tflat = tags.astype(jnp.int32).reshape(NW, BPW, L).transpose(0, 2, 1).reshape(-1)
    scat = jnp.asarray(_SCAT)
    zeros = jnp.zeros((BPW, D), jnp.float32)

    u_e, i_e, c_e, t_e = _sc_embed(
        user_i, item_i, country_i, tflat, scat, zeros,
        user_table, item_table, country_table, tags_table)
    g_e = _genre_emb(genre_i, genre_table)

    return _mlp(u_e, i_e, g_e, c_e, t_e, W1, b1, W2, b2, W3, b3)


# speedup vs baseline: 1.5399x; 1.0044x over previous
import functools

import jax
import jax.numpy as jnp
import numpy as np
from jax import lax
from jax.experimental import pallas as pl
from jax.experimental.pallas import tpu as pltpu
from jax.experimental.pallas import tpu_sc as plsc

B = 4096
D = 128
L = 20
GN = 1000
NC = 2
NS = 16
NW = NC * NS
BPW = B // NW
NBUF = 4


def _sc_embed(user_i, item_i, country_i, tflat, scat, zeros,
              user_table, item_table, country_table, tags_table):
    mesh = plsc.VectorSubcoreMesh(core_axis_name="c", subcore_axis_name="s")
    emb_ty = jax.ShapeDtypeStruct((B, D), jnp.float32)

    @functools.partial(
        pl.kernel,
        mesh=mesh,
        out_type=(emb_ty, emb_ty, emb_ty, emb_ty),
        scratch_types=[
            pltpu.VMEM((3, BPW), jnp.int32),
            pltpu.VMEM((L * BPW,), jnp.int32),
            pltpu.VMEM((8, BPW), jnp.int32),
            pltpu.VMEM((NBUF, BPW, D), jnp.float32),
            pltpu.VMEM_SHARED((NS, BPW, D), jnp.float32),
            pltpu.SemaphoreType.DMA((NBUF,)),
            pltpu.SemaphoreType.DMA((NBUF,)),
            pltpu.SemaphoreType.DMA,
            pltpu.SemaphoreType.DMA((3,)),
            pltpu.SemaphoreType.DMA((3,)),
        ],
    )
    def sc_kernel(u_hbm, i_hbm, c_hbm, tf_hbm, sc_hbm, z_hbm,
                  ut_hbm, it_hbm, ct_hbm, tt_hbm,
                  uo_hbm, io_hbm, co_hbm, to_hbm,
                  idx3_v, idxm_v, scat_v, ring_v, acc_sh,
                  gsem, ssem, zsem, isem, wsem):
        c = lax.axis_index("c")
        s = lax.axis_index("s")
        wid = c * NS + s
        base = wid * BPW

        slab = acc_sh.at[s]
        pltpu.make_async_copy(z_hbm, slab, zsem).start()

        sing = ((u_hbm, ut_hbm, uo_hbm), (i_hbm, it_hbm, io_hbm),
                (c_hbm, ct_hbm, co_hbm))

        def icopy(k):
            return pltpu.make_async_copy(
                sing[k][0].at[pl.ds(base, BPW)], idx3_v.at[k], isem.at[k])

        def sgather(k):
            return pltpu.make_async_copy(
                sing[k][1].at[idx3_v.at[k]], ring_v.at[k], gsem.at[k])

        def swrite(k):
            return pltpu.make_async_copy(
                ring_v.at[k], sing[k][2].at[pl.ds(base, BPW)], wsem.at[k])

        for k in range(3):
            icopy(k).start()
        for k in range(3):
            icopy(k).wait()
            sgather(k).start()
        for k in range(3):
            sgather(k).wait()
            swrite(k).start()

        pltpu.sync_copy(sc_hbm, scat_v)

        def gather(g, buf):
            return pltpu.make_async_copy(
                tt_hbm.at[idxm_v.at[pl.ds(g * BPW, BPW)]],
                ring_v.at[buf], gsem.at[buf])

        def scatter(g, buf):
            return pltpu.make_async_copy(
                ring_v.at[buf], slab.at[scat_v.at[0]], ssem.at[buf])

        pltpu.sync_copy(tf_hbm.at[pl.ds(base * L, L * BPW)], idxm_v)
        pltpu.make_async_copy(z_hbm, slab, zsem).wait()

        for k in range(3):
            swrite(k).wait()

        for g in range(NBUF - 1):
            gather(g, g).start()

        @pl.loop(0, L)
        def _(g):
            buf = lax.rem(g, NBUF)
            gather(g, buf).wait()
            scatter(g, buf).start(add=True)
            nxt = g + NBUF - 1

            @pl.when(nxt < L)
            def _():
                nbuf = lax.rem(nxt, NBUF)
                @pl.when(nxt >= NBUF)
                def _():
                    scatter(nxt - NBUF, nbuf).wait()
                gather(nxt, nbuf).start()

        for g in range(L - NBUF, L):
            scatter(g, g % NBUF).wait()
        pltpu.sync_copy(slab, to_hbm.at[pl.ds(base, BPW)])

    return sc_kernel(user_i, item_i, country_i, tflat, scat, zeros,
                     user_table, item_table, country_table, tags_table)


GTB = 512


def _genre_body(g_ref, gt_ref, o_ref):
    gi = g_ref[...]
    vio = lax.broadcasted_iota(jnp.int32, (GTB, GN), 1)
    counts = (gi[:, 0:1] == vio).astype(jnp.float32)
    for l in range(1, L):
        counts = counts + (gi[:, l:l + 1] == vio).astype(jnp.float32)
    o_ref[...] = jnp.dot(counts, gt_ref[...],
                         preferred_element_type=jnp.float32)


def _genre_emb(genre_i, genre_table):
    return pl.pallas_call(
        _genre_body,
        grid=(B // GTB,),
        in_specs=[
            pl.BlockSpec((GTB, L), lambda n: (n, 0)),
            pl.BlockSpec((GN, D), lambda n: (0, 0)),
        ],
        out_specs=pl.BlockSpec((GTB, D), lambda n: (n, 0)),
        out_shape=jax.ShapeDtypeStruct((B, D), jnp.float32),
    )(genre_i, genre_table)


TB = 512


def _mlp_body(u_ref, i_ref, g_ref, c_ref, t_ref,
              w1_ref, b1_ref, w2_ref, b2_ref, w3_ref, b3_ref, o_ref):
    embs = (u_ref[...], i_ref[...], g_ref[...], c_ref[...], t_ref[...])
    h = b1_ref[...]
    for idx, e in enumerate(embs):
        h = h + jnp.dot(e, w1_ref[idx], preferred_element_type=jnp.float32)
    h = jnp.maximum(h, 0.0)
    h2 = jnp.dot(h, w2_ref[...], preferred_element_type=jnp.float32)
    h2 = jnp.maximum(h2 + b2_ref[...], 0.0)
    o = jnp.dot(h2, w3_ref[...], preferred_element_type=jnp.float32)
    o_ref[...] = o + b3_ref[...]


def _mlp(u, i, g, c, t, W1, b1, W2, b2, W3, b3):
    w1r = W1.reshape(5, D, D)
    full = lambda shape: pl.BlockSpec(shape, lambda n: tuple(0 for _ in shape))
    out = pl.pallas_call(
        _mlp_body,
        grid=(B // TB,),
        in_specs=[
            pl.BlockSpec((TB, D), lambda n: (n, 0)),
            pl.BlockSpec((TB, D), lambda n: (n, 0)),
            pl.BlockSpec((TB, D), lambda n: (n, 0)),
            pl.BlockSpec((TB, D), lambda n: (n, 0)),
            pl.BlockSpec((TB, D), lambda n: (n, 0)),
            full((5, D, D)),
            full((1, D)),
            full((D, D // 2)),
            full((1, D // 2)),
            full((D // 2, 1)),
            full((1, 1)),
        ],
        out_specs=pl.BlockSpec((TB, 1), lambda n: (n, 0)),
        out_shape=jax.ShapeDtypeStruct((B, 1), jnp.float32),
    )(u, i, g, c, t, w1r, b1.reshape(1, D), W2, b2.reshape(1, D // 2),
      W3, b3.reshape(1, 1))
    return out.reshape(-1)


_SCAT = np.tile(np.arange(BPW, dtype=np.int32), (8, 1))


def kernel(user, item, genre, country, tags,
           user_table, item_table, genre_table, country_table, tags_table,
           W1, b1, W2, b2, W3, b3):
    user_i = user.astype(jnp.int32)
    item_i = item.astype(jnp.int32)
    country_i = country.astype(jnp.int32)
    genre_i = genre.astype(jnp.int32)
    tflat = tags.astype(jnp.int32).reshape(NW, BPW, L).transpose(0, 2, 1).reshape(-1)
    scat = jnp.asarray(_SCAT)
    zeros = jnp.zeros((BPW, D), jnp.float32)

    u_e, i_e, c_e, t_e = _sc_embed(
        user_i, item_i, country_i, tflat, scat, zeros,
        user_table, item_table, country_table, tags_table)
    g_e = _genre_emb(genre_i, genre_table)

    return _mlp(u_e, i_e, g_e, c_e, t_e, W1, b1, W2, b2, W3, b3)
